# baseline (speedup 1.0000x reference)
import functools

import jax
import jax.numpy as jnp
from jax import lax
from jax.experimental import pallas as pl
from jax.experimental.pallas import tpu as pltpu
from jax.experimental.pallas import tpu_sc as plsc

_HZ_TO_K = 6.62607015e-34 / 1.380649e-23

_NW = 32
_LANES = 16
_CHUNK = 512


def _make_sc_kernel(B, M, L):
    rows = B * M
    rows_w = rows // _NW
    nchunk = rows_w // _CHUNK
    assert rows_w * _NW == rows and nchunk * _CHUNK == rows_w
    groups = _CHUNK // _LANES
    mesh = plsc.VectorSubcoreMesh(core_axis_name="c", subcore_axis_name="s")

    @functools.partial(
        pl.kernel,
        mesh=mesh,
        compiler_params=pltpu.CompilerParams(needs_layout_passes=False),
        out_type=jax.ShapeDtypeStruct((rows,), jnp.float32),
        scratch_types=[
            pltpu.VMEM((_CHUNK * L,), jnp.float32),
            pltpu.VMEM((_CHUNK * L,), jnp.float32),
            pltpu.VMEM((_CHUNK,), jnp.float32),
            pltpu.VMEM((_CHUNK,), jnp.float32),
            pltpu.VMEM((256,), jnp.int32),
            pltpu.VMEM((256,), jnp.int32),
            pltpu.VMEM((_LANES,), jnp.float32),
            pltpu.SemaphoreType.DMA,
            pltpu.SemaphoreType.DMA,
            pltpu.SemaphoreType.DMA,
            pltpu.SemaphoreType.DMA,
        ],
    )
    def k(e_hbm, down_hbm, up_hbm, scale_hbm, out_hbm,
          buf_a, buf_b, oub_a, oub_b, down_v, up_v, scale_v,
          sem_a, sem_b, sem_oa, sem_ob):
        wid = lax.axis_index("s") * 2 + lax.axis_index("c")
        row0 = wid * rows_w

        pltpu.sync_copy(down_hbm, down_v)
        pltpu.sync_copy(up_hbm, up_v)
        pltpu.sync_copy(scale_hbm, scale_v)
        s = scale_v[...]
        lane = lax.broadcasted_iota(jnp.int32, (_LANES,), 0)

        def start_in(ci, buf, sem):
            pltpu.make_async_copy(
                e_hbm.at[pl.ds((row0 + ci * _CHUNK) * L, _CHUNK * L)],
                buf, sem).start()

        def wait_in(ci, buf, sem):
            pltpu.make_async_copy(
                e_hbm.at[pl.ds((row0 + ci * _CHUNK) * L, _CHUNK * L)],
                buf, sem).wait()

        def start_out(ci, oub, sem):
            pltpu.make_async_copy(
                oub, out_hbm.at[pl.ds(row0 + ci * _CHUNK, _CHUNK)],
                sem).start()

        def wait_out(ci, oub, sem):
            pltpu.make_async_copy(
                oub, out_hbm.at[pl.ds(row0 + ci * _CHUNK, _CHUNK)],
                sem).wait()

        def compute(ci, buf, oub):
            def group(g, _):
                r_loc = g * _LANES + lane
                base = r_loc * L
                m_idx = lax.rem(ci * _CHUNK + r_loc, M)
                dn = plsc.load_gather(down_v, [m_idx])
                up = plsc.load_gather(up_v, [m_idx])
                acc = jnp.exp(plsc.load_gather(buf, [base]) * s)
                for l in range(1, L):
                    acc = acc + jnp.exp(plsc.load_gather(buf, [base + l]) * s)
                e_d = jnp.exp(plsc.load_gather(buf, [base + dn]) * s)
                e_u = jnp.exp(plsc.load_gather(buf, [base + up]) * s)
                oub[pl.ds(g * _LANES, _LANES)] = (e_d - e_u) / acc
                return 0
            lax.fori_loop(0, groups, group, 0)

        start_in(0, buf_a, sem_a)
        start_in(1, buf_b, sem_b)

        def body(i, _):
            ca = 2 * i
            cb = 2 * i + 1
            wait_in(ca, buf_a, sem_a)

            @pl.when(i > 0)
            def _():
                wait_out(ca - 2, oub_a, sem_oa)

            compute(ca, buf_a, oub_a)

            @pl.when(ca + 2 < nchunk)
            def _():
                start_in(ca + 2, buf_a, sem_a)

            start_out(ca, oub_a, sem_oa)

            wait_in(cb, buf_b, sem_b)

            @pl.when(i > 0)
            def _():
                wait_out(cb - 2, oub_b, sem_ob)

            compute(cb, buf_b, oub_b)

            @pl.when(cb + 2 < nchunk)
            def _():
                start_in(cb + 2, buf_b, sem_b)

            start_out(cb, oub_b, sem_ob)
            return 0

        lax.fori_loop(0, nchunk // 2, body, 0)
        wait_out(nchunk - 2, oub_a, sem_oa)
        wait_out(nchunk - 1, oub_b, sem_ob)

    return k


def kernel(energies, lvl_down, lvl_up, temperature):
    B, M, L = energies.shape
    e_flat = energies.reshape(-1)
    down = jnp.pad(lvl_down.astype(jnp.int32), (0, 256 - M))
    up = jnp.pad(lvl_up.astype(jnp.int32), (0, 256 - M))
    scale = jnp.full((_LANES,), -_HZ_TO_K, jnp.float32) / temperature.astype(jnp.float32)
    out = _make_sc_kernel(B, M, L)(e_flat, down, up, scale)
    return out.reshape(B, M)


# speedup vs baseline: 2.3664x; 2.3664x over previous
import functools

import jax
import jax.numpy as jnp
from jax import lax
from jax.experimental import pallas as pl
from jax.experimental.pallas import tpu as pltpu
from jax.experimental.pallas import tpu_sc as plsc

_HZ_TO_K = 6.62607015e-34 / 1.380649e-23

_NW = 32
_LANES = 16
_CB = 2


def _make_sc_kernel(B, M, L):
    b_w = B // _NW
    nchunk = b_w // _CB
    rows_c = _CB * M
    groups = rows_c // _LANES
    assert b_w * _NW == B and nchunk * _CB == b_w and groups * _LANES == rows_c
    mesh = plsc.VectorSubcoreMesh(core_axis_name="c", subcore_axis_name="s")

    @functools.partial(
        pl.kernel,
        mesh=mesh,
        compiler_params=pltpu.CompilerParams(needs_layout_passes=False),
        out_type=jax.ShapeDtypeStruct((B, M), jnp.float32),
        scratch_types=[
            pltpu.VMEM((_CB, M, L), jnp.float32),
            pltpu.VMEM((_CB, M, L), jnp.float32),
            pltpu.VMEM((_CB, M), jnp.float32),
            pltpu.VMEM((_CB, M), jnp.float32),
            pltpu.VMEM((256,), jnp.int32),
            pltpu.VMEM((256,), jnp.int32),
            pltpu.VMEM((_LANES,), jnp.float32),
            pltpu.SemaphoreType.DMA,
            pltpu.SemaphoreType.DMA,
            pltpu.SemaphoreType.DMA,
            pltpu.SemaphoreType.DMA,
        ],
    )
    def k(e_hbm, down_hbm, up_hbm, scale_hbm, out_hbm,
          buf_a, buf_b, oub_a, oub_b, down_v, up_v, scale_v,
          sem_a, sem_b, sem_oa, sem_ob):
        wid = lax.axis_index("s") * 2 + lax.axis_index("c")
        b0 = wid * b_w

        pltpu.sync_copy(down_hbm, down_v)
        pltpu.sync_copy(up_hbm, up_v)
        pltpu.sync_copy(scale_hbm, scale_v)
        s = scale_v[...]
        lane = lax.broadcasted_iota(jnp.int32, (_LANES,), 0)

        def start_in(ci, buf, sem):
            pltpu.make_async_copy(
                e_hbm.at[pl.ds(b0 + ci * _CB, _CB)], buf, sem).start()

        def wait_in(ci, buf, sem):
            pltpu.make_async_copy(
                e_hbm.at[pl.ds(b0 + ci * _CB, _CB)], buf, sem).wait()

        def start_out(ci, oub, sem):
            pltpu.make_async_copy(
                oub, out_hbm.at[pl.ds(b0 + ci * _CB, _CB)], sem).start()

        def wait_out(ci, oub, sem):
            pltpu.make_async_copy(
                oub, out_hbm.at[pl.ds(b0 + ci * _CB, _CB)], sem).wait()

        def compute(buf, oub):
            def group(g, _):
                r = g * _LANES + lane
                bi = lax.div(r, M)
                mi = r - bi * M
                dn = plsc.load_gather(down_v, [mi])
                up = plsc.load_gather(up_v, [mi])
                rot = lane
                acc = jnp.exp(plsc.load_gather(buf, [bi, mi, rot]) * s)
                for _l in range(1, L):
                    rot = (rot + 1) & (L - 1)
                    acc = acc + jnp.exp(plsc.load_gather(buf, [bi, mi, rot]) * s)
                e_d = jnp.exp(plsc.load_gather(buf, [bi, mi, dn]) * s)
                e_u = jnp.exp(plsc.load_gather(buf, [bi, mi, up]) * s)
                plsc.store_scatter(oub, [bi, mi], (e_d - e_u) / acc)
                return 0
            lax.fori_loop(0, groups, group, 0)

        start_in(0, buf_a, sem_a)
        start_in(1, buf_b, sem_b)

        def body(i, _):
            ca = 2 * i
            cb = 2 * i + 1
            wait_in(ca, buf_a, sem_a)

            @pl.when(i > 0)
            def _():
                wait_out(ca - 2, oub_a, sem_oa)

            compute(buf_a, oub_a)

            @pl.when(ca + 2 < nchunk)
            def _():
                start_in(ca + 2, buf_a, sem_a)

            start_out(ca, oub_a, sem_oa)

            wait_in(cb, buf_b, sem_b)

            @pl.when(i > 0)
            def _():
                wait_out(cb - 2, oub_b, sem_ob)

            compute(buf_b, oub_b)

            @pl.when(cb + 2 < nchunk)
            def _():
                start_in(cb + 2, buf_b, sem_b)

            start_out(cb, oub_b, sem_ob)
            return 0

        lax.fori_loop(0, nchunk // 2, body, 0)
        wait_out(nchunk - 2, oub_a, sem_oa)
        wait_out(nchunk - 1, oub_b, sem_ob)

    return k


def kernel(energies, lvl_down, lvl_up, temperature):
    B, M, L = energies.shape
    down = jnp.pad(lvl_down.astype(jnp.int32), (0, 256 - M))
    up = jnp.pad(lvl_up.astype(jnp.int32), (0, 256 - M))
    scale = jnp.full((_LANES,), -_HZ_TO_K, jnp.float32) / temperature.astype(jnp.float32)
    return _make_sc_kernel(B, M, L)(energies, down, up, scale)
